# Initial kernel scaffold; baseline (speedup 1.0000x reference)
import jax
import jax.numpy as jnp
from jax.experimental import pallas as pl


def kernel(feat, edge_index, W1, b1, W2, b2, temp):
    raise NotImplementedError("write your pallas kernel here")


# speedup vs baseline: 15.1265x; 15.1265x over previous
import functools

import jax
import jax.numpy as jnp
from jax import lax
from jax.experimental import pallas as pl
from jax.experimental.pallas import tpu as pltpu
from jax.experimental.pallas import tpu_sc as plsc

N = 10000
E = 320000
D = 128
H = 64
C = 40
CP = 48
K = 10
NT = 32
EPT = E // NT
G = 128
NCH = (EPT + G - 1) // G
EPAD = NCH * G
TRASH = N
NP = 10240
RPT = NP // 16
RCH = 5
RB = RPT // RCH
DEGN = 10240
DPT = DEGN // 16
DW = 8

_mesh = plsc.VectorSubcoreMesh(core_axis_name="c", subcore_axis_name="s")
_sc_params = pltpu.CompilerParams(use_tc_tiling_on_sc=False)


@functools.partial(
    pl.kernel,
    out_type=jax.ShapeDtypeStruct((2 * DEGN, DW), jnp.float32),
    mesh=_mesh,
    scratch_types=[
        pltpu.VMEM((G, DW), jnp.float32),
        pltpu.VMEM((DPT, DW), jnp.float32),
        pltpu.VMEM((NCH, G), jnp.int32),
        pltpu.VMEM_SHARED((DEGN, DW), jnp.float32),
    ],
    compiler_params=_sc_params,
)
def _deg_kernel(dsts, ones, zeros, out, ones_v, stage_v, dst_v, acc):
    c = lax.axis_index("c")
    s = lax.axis_index("s")
    g = c * 16 + s
    pltpu.sync_copy(dsts.at[g], dst_v)
    pltpu.sync_copy(ones, ones_v)
    pltpu.sync_copy(zeros, stage_v)
    pltpu.sync_copy(stage_v, acc.at[pl.ds(s * DPT, DPT)])
    plsc.subcore_barrier()

    def ebody(j, carry):
        pltpu.sync_copy(ones_v, acc.at[dst_v.at[j]], add=True)
        return carry

    lax.fori_loop(0, NCH, ebody, 0)
    plsc.subcore_barrier()
    pltpu.sync_copy(acc.at[pl.ds(s * DPT, DPT)], stage_v)
    pltpu.sync_copy(stage_v, out.at[pl.ds(c * DEGN + s * DPT, DPT)])


@functools.partial(
    pl.kernel,
    out_type=(
        jax.ShapeDtypeStruct((2 * NP, CP), jnp.float32),
        jax.ShapeDtypeStruct((NP, CP), jnp.float32),
        jax.ShapeDtypeStruct((2 * NP, CP), jnp.float32),
    ),
    mesh=_mesh,
    scratch_types=[
        pltpu.VMEM((16,), jnp.float32),
        pltpu.VMEM((NCH, G), jnp.int32),
        pltpu.VMEM((NCH, G), jnp.int32),
        pltpu.VMEM((RB, CP), jnp.float32),
        pltpu.VMEM((RB, CP), jnp.float32),
        pltpu.VMEM((RB, CP), jnp.float32),
        pltpu.VMEM((RB, CP), jnp.float32),
        pltpu.VMEM((RB, CP), jnp.float32),
        pltpu.VMEM((RB, CP), jnp.float32),
        pltpu.VMEM((G, CP), jnp.float32),
        pltpu.VMEM_SHARED((NP, CP), jnp.float32),
    ],
    compiler_params=_sc_params,
)
def _round_kernel(p, rinv, t_in, tk, srcs, dsts, q, t_out, zin_hbm,
                  tk_v, src_v, dst_v, a_v, b_v, r_v, t_v, zi_v, h_v, g_v, acc):
    c = lax.axis_index("c")
    s = lax.axis_index("s")
    g = c * 16 + s
    pltpu.sync_copy(tk, tk_v)
    pltpu.sync_copy(srcs.at[g], src_v)
    pltpu.sync_copy(dsts.at[g], dst_v)
    tks = tk_v[...]

    for cc in range(RCH):
        row0 = s * RPT + cc * RB
        pltpu.sync_copy(p.at[pl.ds(row0, RB)], a_v)
        pltpu.sync_copy(p.at[pl.ds(NP + row0, RB)], b_v)
        pltpu.sync_copy(rinv.at[pl.ds(row0, RB)], r_v)
        pltpu.sync_copy(t_in.at[pl.ds(row0, RB)], t_v)

        def rowbody(i, carry):
            for q3 in range(CP // 16):
                sl = pl.ds(q3 * 16, 16)
                z = a_v[i, sl] + b_v[i, sl]
                zi = z * r_v[i, sl]
                zi_v[i, sl] = zi
                h_v[i, sl] = zi * 0.5
                t_v[i, sl] = t_v[i, sl] + tks * z
            return carry

        lax.fori_loop(0, RB, rowbody, 0)
        pltpu.sync_copy(zi_v, zin_hbm.at[pl.ds(c * NP + row0, RB)])
        pltpu.sync_copy(h_v, acc.at[pl.ds(row0, RB)])

        @pl.when(c == 0)
        def _():
            pltpu.sync_copy(t_v, t_out.at[pl.ds(row0, RB)])

    plsc.subcore_barrier()

    def ebody(j, carry):
        pltpu.sync_copy(zin_hbm.at[src_v.at[j]], g_v)
        pltpu.sync_copy(g_v, acc.at[dst_v.at[j]], add=True)
        return carry

    lax.fori_loop(0, NCH, ebody, 0)
    plsc.subcore_barrier()

    for cc in range(RCH):
        row0 = s * RPT + cc * RB
        pltpu.sync_copy(acc.at[pl.ds(row0, RB)], zi_v)
        pltpu.sync_copy(zi_v, q.at[pl.ds(c * NP + row0, RB)])


_BLK = 1000


def _prep_body(feat_ref, w1_ref, b1_ref, w2_ref, b2_ref, degp_ref,
               z0_ref, r_ref):
    h1 = jnp.dot(feat_ref[...], w1_ref[...],
                 preferred_element_type=jnp.float32) + b1_ref[...][None, :]
    h1 = jnp.maximum(h1, 0.0)
    h = jnp.dot(h1, w2_ref[...],
                preferred_element_type=jnp.float32) + b2_ref[...][None, :]
    deg = degp_ref[:, 0] + degp_ref[:, 1] + 1.0
    z0 = h * jnp.sqrt(deg)[:, None]
    z0_ref[...] = jnp.concatenate(
        [z0, jnp.zeros((_BLK, CP - C), jnp.float32)], axis=1)
    r_ref[...] = jnp.broadcast_to((1.0 / deg)[:, None], (_BLK, CP))


def _prep(feat, W1, b1, W2, b2, degp2):
    return pl.pallas_call(
        _prep_body,
        grid=(N // _BLK,),
        in_specs=[
            pl.BlockSpec((_BLK, D), lambda i: (i, 0)),
            pl.BlockSpec((D, H), lambda i: (0, 0)),
            pl.BlockSpec((H,), lambda i: (0,)),
            pl.BlockSpec((H, C), lambda i: (0, 0)),
            pl.BlockSpec((C,), lambda i: (0,)),
            pl.BlockSpec((_BLK, 2), lambda i: (i, 0)),
        ],
        out_specs=[
            pl.BlockSpec((_BLK, CP), lambda i: (i, 0)),
            pl.BlockSpec((_BLK, CP), lambda i: (i, 0)),
        ],
        out_shape=[
            jax.ShapeDtypeStruct((N, CP), jnp.float32),
            jax.ShapeDtypeStruct((N, CP), jnp.float32),
        ],
    )(feat, W1, b1, W2, b2, degp2)


def _final_body(temp_ref, t_ref, q0_ref, q1_ref, degp_ref, out_ref):
    deg = degp_ref[:, 0] + degp_ref[:, 1] + 1.0
    dinv = lax.rsqrt(deg)[:, None]
    th = (t_ref[...] + temp_ref[0, K] * (q0_ref[...] + q1_ref[...])) * dinv
    x = th[:, :C]
    m = jnp.max(x, axis=1, keepdims=True)
    ex = jnp.exp(x - m)
    lse = jnp.log(jnp.sum(ex, axis=1, keepdims=True)) + m
    out_ref[...] = x - lse


def _final(temp2, t_acc, q0, q1, degp2):
    return pl.pallas_call(
        _final_body,
        grid=(N // _BLK,),
        in_specs=[
            pl.BlockSpec((1, 16), lambda i: (0, 0)),
            pl.BlockSpec((_BLK, CP), lambda i: (i, 0)),
            pl.BlockSpec((_BLK, CP), lambda i: (i, 0)),
            pl.BlockSpec((_BLK, CP), lambda i: (i, 0)),
            pl.BlockSpec((_BLK, 2), lambda i: (i, 0)),
        ],
        out_specs=pl.BlockSpec((_BLK, C), lambda i: (i, 0)),
        out_shape=jax.ShapeDtypeStruct((N, C), jnp.float32),
    )(temp2, t_acc, q0, q1, degp2)


def kernel(feat, edge_index, W1, b1, W2, b2, temp):
    src = edge_index[0].reshape(NT, EPT)
    dst = edge_index[1].reshape(NT, EPT)
    src = jnp.pad(src, ((0, 0), (0, EPAD - EPT)))
    dst = jnp.pad(dst, ((0, 0), (0, EPAD - EPT)), constant_values=TRASH)
    shift = (jnp.arange(NT, dtype=jnp.int32) // 16 * NP)[:, None]
    src3 = (src + shift).reshape(NT, NCH, G)
    dst3 = dst.reshape(NT, NCH, G)

    ones = jnp.ones((G, DW), jnp.float32)
    zeros = jnp.zeros((DPT, DW), jnp.float32)
    degp = _deg_kernel(dst3, ones, zeros)
    degp2 = jnp.stack([degp[:N, 0], degp[DEGN:DEGN + N, 0]], axis=1)

    z0, rinv = _prep(feat, W1, b1, W2, b2, degp2)
    temp2 = jnp.pad(temp, (0, 16 - (K + 1))).reshape(1, 16)

    z0p = jnp.pad(z0, ((0, NP - N), (0, 0)))
    rinvp = jnp.pad(rinv, ((0, NP - N), (0, 0)))
    q = jnp.concatenate([z0p, jnp.zeros((NP, CP), jnp.float32)], axis=0)
    t = jnp.zeros((NP, CP), jnp.float32)
    for k in range(K):
        tk16 = jnp.broadcast_to(temp2[0, k], (16,))
        q, t, _ = _round_kernel(q, rinvp, t, tk16, src3, dst3)

    return _final(temp2, t[:N], q[:N], q[NP:NP + N], degp2)
